# Initial kernel scaffold; baseline (speedup 1.0000x reference)
import jax
import jax.numpy as jnp
from jax.experimental import pallas as pl


def kernel(edges, feature_a, feature_b, node_num_a, node_num_b, W, b, a):
    raise NotImplementedError("write your pallas kernel here")


# speedup vs baseline: 15.0069x; 15.0069x over previous
import functools

import jax
import jax.numpy as jnp
from jax import lax
from jax.experimental import pallas as pl
from jax.experimental.pallas import tpu as pltpu
from jax.experimental.pallas import tpu_sc as plsc

NC = 2
NS = 16
L = 16


def _pre_body(fb_ref, fa_ref, w_ref, b_ref, aa_ref, ab_ref,
              ne_ref, sa_ref, sb_ref):
    fb = fb_ref[...]
    ne = lax.dot_general(fb, w_ref[...], (((1,), (1,)), ((), ())),
                         preferred_element_type=jnp.float32) + b_ref[...]
    ne_ref[...] = ne
    sa_ref[...] = lax.dot_general(aa_ref[...], fa_ref[...],
                                  (((1,), (1,)), ((), ())),
                                  preferred_element_type=jnp.float32)
    sb_ref[...] = lax.dot_general(ab_ref[...], ne,
                                  (((1,), (1,)), ((), ())),
                                  preferred_element_type=jnp.float32)


def _precompute(feature_b, feature_a, W, b, aa, ab, blk):
    n, d = feature_b.shape
    grid = ((n + blk - 1) // blk,)
    return pl.pallas_call(
        _pre_body,
        grid=grid,
        in_specs=[
            pl.BlockSpec((blk, d), lambda i: (i, 0)),
            pl.BlockSpec((blk, d), lambda i: (i, 0)),
            pl.BlockSpec((d, d), lambda i: (0, 0)),
            pl.BlockSpec((1, d), lambda i: (0, 0)),
            pl.BlockSpec((1, d), lambda i: (0, 0)),
            pl.BlockSpec((1, d), lambda i: (0, 0)),
        ],
        out_specs=[
            pl.BlockSpec((blk, d), lambda i: (i, 0)),
            pl.BlockSpec((1, blk), lambda i: (0, i)),
            pl.BlockSpec((1, blk), lambda i: (0, i)),
        ],
        out_shape=[
            jax.ShapeDtypeStruct((n, d), jnp.float32),
            jax.ShapeDtypeStruct((1, n), jnp.float32),
            jax.ShapeDtypeStruct((1, n), jnp.float32),
        ],
    )(feature_b, feature_a, W, b, aa, ab)


def _make_sc_agg(E, Na, D, C, nap, nb):
    EP = E // (NC * NS)
    NCH = EP // C
    RPT = nap // NS
    ZR = 128
    RST = nap // NS
    GC = C // L

    mesh = plsc.VectorSubcoreMesh(core_axis_name="c", subcore_axis_name="s",
                                  num_cores=NC, num_subcores=NS)

    @functools.partial(
        pl.kernel,
        out_type=(
            jax.ShapeDtypeStruct((NC * nap, D), jnp.float32),
            jax.ShapeDtypeStruct((NC * nap,), jnp.float32),
        ),
        mesh=mesh,
        compiler_params=pltpu.CompilerParams(needs_layout_passes=False),
        scratch_types=[
            pltpu.VMEM((nb, C), jnp.int32),
            pltpu.VMEM((nb, C), jnp.int32),
            pltpu.VMEM((nb, C, D), jnp.float32),
            pltpu.VMEM((nb, C), jnp.float32),
            pltpu.VMEM((nb, C), jnp.float32),
            pltpu.VMEM((nb, C), jnp.float32),
            pltpu.VMEM_SHARED((nap, D), jnp.float32),
            pltpu.VMEM_SHARED((nap,), jnp.float32),
            pltpu.VMEM_SHARED((Na,), jnp.float32),
            pltpu.VMEM_SHARED((Na,), jnp.float32),
            pltpu.SemaphoreType.DMA((nb,)),
            pltpu.SemaphoreType.DMA((nb,)),
            pltpu.SemaphoreType.DMA((nb,)),
        ],
    )
    def _sc(src_hbm, dst_hbm, sa_hbm, sb_hbm, ne_hbm, out_hbm, rs_hbm,
            src_v, dst_v, rows_v, w_v, sas_v, sbs_v, acc, rs_acc,
            sa_sh, sb_sh, isem, gsem, ssem):
        cc = lax.axis_index("c")
        ss = lax.axis_index("s")
        wid = ss * NC + cc

        def stage_idx(i, b):
            base = wid * EP + i * C
            pltpu.async_copy(src_hbm.at[pl.ds(base, C)], src_v.at[b],
                             isem.at[b])
            pltpu.async_copy(dst_hbm.at[pl.ds(base, C)], dst_v.at[b],
                             isem.at[b])

        def wait_idx(b):
            pltpu.make_async_copy(src_hbm.at[pl.ds(0, C)], src_v.at[b],
                                  isem.at[b]).wait()
            pltpu.make_async_copy(src_hbm.at[pl.ds(0, C)], dst_v.at[b],
                                  isem.at[b]).wait()

        def start_gathers(b):
            pltpu.async_copy(ne_hbm.at[dst_v.at[b]], rows_v.at[b],
                             gsem.at[b])
            pltpu.async_copy(sa_sh.at[src_v.at[b]], sas_v.at[b], isem.at[b])
            pltpu.async_copy(sb_sh.at[dst_v.at[b]], sbs_v.at[b], isem.at[b])

        def wait_gathers(b):
            pltpu.make_async_copy(ne_hbm.at[pl.ds(0, C)], rows_v.at[b],
                                  gsem.at[b]).wait()
            pltpu.make_async_copy(sa_hbm.at[pl.ds(0, C)], sas_v.at[b],
                                  isem.at[b]).wait()
            pltpu.make_async_copy(sa_hbm.at[pl.ds(0, C)], sbs_v.at[b],
                                  isem.at[b]).wait()

        def start_scatter(b):
            pltpu.async_copy(rows_v.at[b], acc.at[src_v.at[b]],
                             ssem.at[b], add=True)
            pltpu.async_copy(w_v.at[b], rs_acc.at[src_v.at[b]], ssem.at[b],
                             add=True)

        def wait_scatter(b):
            pltpu.make_async_copy(out_hbm.at[pl.ds(0, C)], rows_v.at[b],
                                  ssem.at[b]).wait()
            pltpu.make_async_copy(sa_hbm.at[pl.ds(0, C)], w_v.at[b],
                                  ssem.at[b]).wait()

        def compute(b):
            @pl.loop(0, GC)
            def _grp(g):
                sl = pl.ds(g * L, L)
                x = sas_v[b, sl] + sbs_v[b, sl]
                ex = jnp.exp(x)
                w16 = jnp.exp(jnp.where(x > 0.0, x, 0.1 * ex - 0.1))
                w_v[b, sl] = w16
                for lane in range(L):
                    wb = w16.at[jnp.full((L,), lane, jnp.int32)].get(
                        mode="promise_in_bounds")
                    j = g * L + lane
                    for k in range(D // L):
                        ksl = pl.ds(k * L, L)
                        rows_v[b, j, ksl] = rows_v[b, j, ksl] * wb

        zero16 = jnp.zeros((L,), jnp.float32)

        @pl.loop(0, C)
        def _zrow(r):
            for k in range(D // L):
                rows_v[0, r, pl.ds(k * L, L)] = zero16
        for g in range(GC):
            w_v[0, pl.ds(g * L, L)] = zero16

        for j in range(RPT // C):
            pltpu.sync_copy(rows_v.at[0], acc.at[pl.ds(ss * RPT + j * C, C)])
        for j in range(RST // C):
            pltpu.sync_copy(w_v.at[0],
                            rs_acc.at[pl.ds(ss * RST + j * C, C)])

        @pl.when(ss == 0)
        def _():
            pltpu.sync_copy(sa_hbm, sa_sh)
            pltpu.sync_copy(sb_hbm, sb_sh)

        plsc.subcore_barrier()

        stage_idx(0, 0)
        stage_idx(1, 1)
        wait_idx(0)
        start_gathers(0)

        assert nb == 4 and NCH % nb == 1

        @pl.loop(0, NCH // nb)
        def _round(r):
            i0 = r * nb
            for b in range(nb):
                i = i0 + b
                b1 = (b + 1) % nb
                b2 = (b + 2) % nb

                @pl.when(i >= 2)
                def _():
                    wait_scatter(b2)

                @pl.when(i + 2 < NCH)
                def _():
                    stage_idx(i + 2, b2)

                @pl.when(i + 1 < NCH)
                def _():
                    wait_idx(b1)
                    start_gathers(b1)

                wait_gathers(b)
                compute(b)
                start_scatter(b)

        wait_scatter(2)
        wait_gathers(0)
        compute(0)
        start_scatter(0)

        wait_scatter(3)
        wait_scatter(0)

        plsc.subcore_barrier()

        for j in range(RPT // ZR):
            r0 = ss * RPT + j * ZR
            pltpu.sync_copy(acc.at[pl.ds(r0, ZR)],
                            out_hbm.at[pl.ds(cc * nap + r0, ZR)])
        for j in range(RST // D):
            r0 = ss * RST + j * D
            pltpu.sync_copy(rs_acc.at[pl.ds(r0, D)],
                            rs_hbm.at[pl.ds(cc * nap + r0, D)])

    return _sc


def _fin_body(p_ref, r_ref, o_ref):
    rs = r_ref[0] + r_ref[1]
    rs = jnp.where(rs == 0.0, 1.0, rs)
    o_ref[...] = (p_ref[0] + p_ref[1]) / rs


def _combine(partials3, rs3, Na, nap, D, blk):
    return pl.pallas_call(
        _fin_body,
        grid=(Na // blk,),
        in_specs=[
            pl.BlockSpec((2, blk, D), lambda i: (0, i, 0)),
            pl.BlockSpec((2, blk, 1), lambda i: (0, i, 0)),
        ],
        out_specs=pl.BlockSpec((blk, D), lambda i: (i, 0)),
        out_shape=jax.ShapeDtypeStruct((Na, D), jnp.float32),
    )(partials3, rs3)


def kernel(edges, feature_a, feature_b, node_num_a, node_num_b, W, b, a):
    Na, a_dim = feature_a.shape
    Nb, b_dim = feature_b.shape
    E = edges.shape[0]
    D = b_dim

    src = edges[:, 0].astype(jnp.int32)
    dst = edges[:, 1].astype(jnp.int32)
    aa = a[:a_dim].astype(jnp.float32).reshape(1, a_dim)
    ab = a[a_dim:].astype(jnp.float32).reshape(1, b_dim)
    b2 = b.astype(jnp.float32).reshape(1, D)

    ne, sa, sb = _precompute(feature_b.astype(jnp.float32),
                             feature_a.astype(jnp.float32),
                             W.astype(jnp.float32), b2, aa, ab, blk=1024)

    nap = ((Na + NS * D - 1) // (NS * D)) * (NS * D)
    sc_agg = _make_sc_agg(E, Na, D, C=80, nap=nap, nb=4)
    partials, rs_flat = sc_agg(src, dst, sa.reshape(Na), sb.reshape(Na), ne)

    return _combine(partials.reshape(NC, nap, D),
                    rs_flat.reshape(NC, nap, 1), Na, nap, D, blk=1000)
